# Initial kernel scaffold; baseline (speedup 1.0000x reference)
import jax
import jax.numpy as jnp
from jax.experimental import pallas as pl


def kernel(x, edge_attr, params, edge_index, batch):
    raise NotImplementedError("write your pallas kernel here")


# speedup vs baseline: 2.2031x; 2.2031x over previous
import functools

import jax
import jax.numpy as jnp
from jax import lax
from jax.experimental import pallas as pl
from jax.experimental.pallas import tpu as pltpu
from jax.experimental.pallas import tpu_sc as plsc

_EMB = 128
_N = 10000
_E = 320000
_NG = 128
_GAMMA = 0.4
_NS = 16
_CH = 80
_EPT = _E // _NS
_NCHUNK = _EPT // _CH
_NPAD = 10240
_RPT = _NPAD // _NS


def _edge_stage_pair(h_pair, e_pair, src, dst, zeros_tile):
    h_flat = h_pair.reshape(2 * _N, _EMB)
    e_flat = e_pair.reshape(2 * _E, _EMB)
    mesh = plsc.VectorSubcoreMesh(core_axis_name="c", subcore_axis_name="s")

    @functools.partial(
        pl.kernel,
        out_type=jax.ShapeDtypeStruct((2, _NPAD, _EMB), jnp.float32),
        mesh=mesh,
        scratch_types=[
            pltpu.VMEM_SHARED((_NPAD, _EMB), jnp.float32),
            pltpu.VMEM((_CH,), jnp.int32),
            pltpu.VMEM((1, _CH), jnp.int32),
            pltpu.VMEM((1, _CH), jnp.int32),
            pltpu.VMEM((_CH, _EMB), jnp.float32),
            pltpu.VMEM((_CH, _EMB), jnp.float32),
            pltpu.SemaphoreType.DMA,
            pltpu.SemaphoreType.DMA,
        ],
    )
    def k(h_hbm, e_hbm, src_hbm, dst_hbm, z_hbm, out_hbm,
          agg_sh, srcraw_v, src_v, dst_v, rows_v, e_v, sem1, sem2):
        c = lax.axis_index("c")
        s = lax.axis_index("s")
        pltpu.sync_copy(z_hbm, agg_sh.at[pl.ds(s * _RPT, _RPT)])
        plsc.subcore_barrier()
        cN = c * _N

        def chunk_body(g, carry):
            base = s * _EPT + g * _CH
            pltpu.sync_copy(src_hbm.at[pl.ds(base, _CH)], srcraw_v)
            pltpu.sync_copy(dst_hbm.at[pl.ds(base, _CH)], dst_v.at[0])
            for kk in range(_CH // 16):
                sl = pl.ds(kk * 16, 16)
                src_v[0, sl] = srcraw_v[sl] + cN
            gath = pltpu.async_copy(h_hbm.at[src_v.at[0]], rows_v, sem1)
            estr = pltpu.async_copy(e_hbm.at[pl.ds(c * _E + base, _CH)], e_v, sem2)
            gath.wait()
            estr.wait()

            def row_body(r, rc):
                for j in range(_EMB // 16):
                    sl = pl.ds(j * 16, 16)
                    rows_v[r, sl] = jnp.maximum(rows_v[r, sl] + e_v[r, sl], 0.0)
                return rc

            lax.fori_loop(0, _CH, row_body, 0)
            pltpu.sync_copy(rows_v, agg_sh.at[dst_v.at[0]], add=True)
            return carry

        lax.fori_loop(0, _NCHUNK, chunk_body, 0)
        plsc.subcore_barrier()
        pltpu.sync_copy(agg_sh.at[pl.ds(s * _RPT, _RPT)],
                        out_hbm.at[c, pl.ds(s * _RPT, _RPT)])

    return k(h_flat, e_flat, src, dst, zeros_tile)


def _bn(x):
    m = x.mean(axis=0)
    v = x.var(axis=0)
    return (x - m) / jnp.sqrt(v + 1e-5)


def _node_mlp(h, agg, p, relu_after):
    z = (1.0 + p["eps"]) * h + agg
    t = jax.nn.relu(_bn(z @ p["mlp1"]["w"] + p["mlp1"]["b"]))
    u = _bn(t @ p["mlp2"]["w"] + p["mlp2"]["b"])
    if relu_after:
        u = jax.nn.relu(u)
    return u + h


def kernel(x, edge_attr, params, edge_index, batch):
    n = x.shape[0]
    gkey = jax.random.key(42)
    g_noise = jax.random.gumbel(jax.random.fold_in(gkey, 1), (n, 2), dtype=jnp.float32)
    perm = jax.random.permutation(jax.random.fold_in(gkey, 2), _NG)
    zeros_tile = jnp.zeros((_RPT, _EMB), jnp.float32)
    src = edge_index[0].astype(jnp.int32)
    dst = edge_index[1].astype(jnp.int32)

    enc, rat = params["enc_layers"], params["rat_layers"]
    xf = x @ params["node_enc"]["w"] + params["node_enc"]["b"]

    def e_of(p):
        return edge_attr @ p["edge"]["w"] + p["edge"]["b"]

    e1 = jnp.stack([e_of(enc[0]), e_of(rat[0])])
    agg1 = _edge_stage_pair(jnp.stack([xf, xf]), e1, src, dst, zeros_tile)[:, :_N]
    h_enc1 = _node_mlp(xf, agg1[0], enc[0], True)
    h_rat1 = _node_mlp(xf, agg1[1], rat[0], True)
    e2 = jnp.stack([e_of(enc[1]), e_of(rat[1])])
    agg2 = _edge_stage_pair(jnp.stack([h_enc1, h_rat1]), e2, src, dst, zeros_tile)[:, :_N]
    h_node = _node_mlp(h_enc1, agg2[0], enc[1], False)
    x_rat = _node_mlp(h_rat1, agg2[1], rat[1], False)

    gl = (jax.nn.relu(_bn(x_rat @ params["gate1"]["w"] + params["gate1"]["b"]))
          @ params["gate2"]["w"] + params["gate2"]["b"])
    gate = jax.nn.softmax(gl + g_noise, axis=-1)[:, -1:]

    onehot = (batch[None, :] == jnp.arange(_NG)[:, None]).astype(jnp.float32)
    V = jnp.concatenate([h_node, gate * h_node, gate,
                         jnp.ones((n, 1), jnp.float32)], axis=1)
    S = onehot @ V
    sum_h, sum_gh = S[:, :_EMB], S[:, _EMB:2 * _EMB]
    r_sum, cnt = S[:, 2 * _EMB:2 * _EMB + 1], S[:, 2 * _EMB + 1:]
    cnt_c = jnp.maximum(cnt, 1.0)
    h_r = sum_gh / cnt_c
    h_env = (sum_h - sum_gh) / cnt_c
    h_out = sum_h / cnt_c
    r_num = r_sum + 1e-8
    e_num = (cnt - r_sum) + 1e-8
    loss_reg = jnp.abs(r_num / (r_num + e_num) - _GAMMA).mean()

    def head(v):
        return (jax.nn.relu(_bn(v @ params["pred1"]["w"] + params["pred1"]["b"]))
                @ params["pred2"]["w"] + params["pred2"]["b"])

    pred_rem = head(h_r)
    T = 0.2
    xa = jnp.linalg.norm(h_r, axis=1)
    aa = jnp.linalg.norm(h_out, axis=1)
    ca = jnp.linalg.norm(h_env, axis=1)
    sim = jnp.exp((h_r @ h_out.T) / (jnp.outer(xa, aa) + 1e-8) / T)
    sim_cp = jnp.exp((h_r @ h_env.T) / (jnp.outer(xa, ca) + 1e-8) / T)
    pos = jnp.diag(sim)
    loss2 = pos / (sim_cp.sum(axis=1) + pos)
    loss_con = -jnp.log(loss2).mean()
    pred_rep = head(h_r + h_env[perm])
    return (pred_rep, pred_rem, loss_reg, loss_con)


# speedup vs baseline: 2.5103x; 1.1394x over previous
import functools

import jax
import jax.numpy as jnp
from jax import lax
from jax.experimental import pallas as pl
from jax.experimental.pallas import tpu as pltpu
from jax.experimental.pallas import tpu_sc as plsc

_EMB = 128
_N = 10000
_E = 320000
_NG = 128
_GAMMA = 0.4
_NS = 16
_CH = 64
_CPT = 320
_EPT = _E // _NS
_EPT_P = _CH * _CPT
_EPAD = _NS * _EPT_P
_NPAD = 10240
_RPT = _NPAD // _NS


def _edge_stage_pair(h_pair, e_pair, src2, dstf, zeros_tile):
    h_flat = h_pair.reshape(2 * _N, _EMB)
    e_flat = e_pair.reshape(2 * _EPAD, _EMB)
    mesh = plsc.VectorSubcoreMesh(core_axis_name="c", subcore_axis_name="s")

    @functools.partial(
        pl.kernel,
        out_type=jax.ShapeDtypeStruct((2, _NPAD, _EMB), jnp.float32),
        mesh=mesh,
        scratch_types=[
            pltpu.VMEM_SHARED((_NPAD, _EMB), jnp.float32),
            pltpu.VMEM((4, _CH), jnp.int32),
            pltpu.VMEM((4, _CH), jnp.int32),
            pltpu.VMEM((2, _CH, _EMB), jnp.float32),
            pltpu.VMEM((2, _CH, _EMB), jnp.float32),
        ] + [pltpu.SemaphoreType.DMA] * 8,
    )
    def k(h_hbm, e_hbm, src_hbm, dst_hbm, z_hbm, out_hbm,
          agg_sh, srcb, dstb, rows_v, e_v,
          si0, si1, si2, si3, sg0, sg1, se0, se1):
        c = lax.axis_index("c")
        s = lax.axis_index("s")
        si = (si0, si1, si2, si3)
        sg = (sg0, sg1)
        se = (se0, se1)
        ebase = c * _EPAD + s * _EPT_P
        dbase = s * _EPT_P

        pltpu.sync_copy(z_hbm, agg_sh.at[pl.ds(s * _RPT, _RPT)])
        plsc.subcore_barrier()

        def idx_issue(g, slot):
            pltpu.async_copy(src_hbm.at[pl.ds(ebase + g * _CH, _CH)],
                             srcb.at[slot], si[slot])
            pltpu.async_copy(dst_hbm.at[pl.ds(dbase + g * _CH, _CH)],
                             dstb.at[slot], si[slot])

        def idx_wait(slot):
            pltpu.make_async_copy(src_hbm.at[pl.ds(0, _CH)],
                                  srcb.at[slot], si[slot]).wait()
            pltpu.make_async_copy(src_hbm.at[pl.ds(0, _CH)],
                                  dstb.at[slot], si[slot]).wait()

        def ge_issue(g, slot, b):
            pltpu.async_copy(h_hbm.at[srcb.at[slot]], rows_v.at[b], sg[b])
            pltpu.async_copy(e_hbm.at[pl.ds(ebase + g * _CH, _CH)],
                             e_v.at[b], se[b])

        def finish(slot, b):
            pltpu.make_async_copy(h_hbm.at[srcb.at[slot]],
                                  rows_v.at[b], sg[b]).wait()
            pltpu.make_async_copy(e_hbm.at[pl.ds(0, _CH)],
                                  e_v.at[b], se[b]).wait()

            def rg_body(r, rc):
                r4 = r * 4
                for rr in range(4):
                    for j in range(_EMB // 16):
                        sl = pl.ds(j * 16, 16)
                        rows_v[b, r4 + rr, sl] = jnp.maximum(
                            rows_v[b, r4 + rr, sl] + e_v[b, r4 + rr, sl], 0.0)
                return rc

            lax.fori_loop(0, _CH // 4, rg_body, 0)
            pltpu.sync_copy(rows_v.at[b], agg_sh.at[dstb.at[slot]], add=True)

        idx_issue(0, 0)
        idx_issue(1, 1)
        idx_wait(0)
        ge_issue(0, 0, 0)

        def body(i, carry):
            g0 = i * 4
            for u in range(4):
                g = g0 + u

                @pl.when(g + 1 < _CPT)
                def _(u=u, g=g):
                    idx_wait((u + 1) % 4)
                    ge_issue(g + 1, (u + 1) % 4, (u + 1) % 2)

                @pl.when(g + 2 < _CPT)
                def _(u=u, g=g):
                    idx_issue(g + 2, (u + 2) % 4)

                finish(u, u % 2)
            return carry

        lax.fori_loop(0, _CPT // 4, body, 0)
        plsc.subcore_barrier()
        pltpu.sync_copy(agg_sh.at[pl.ds(s * _RPT, _RPT)],
                        out_hbm.at[c, pl.ds(s * _RPT, _RPT)])

    return k(h_flat, e_flat, src2, dstf, zeros_tile)


def _bn(x):
    m = x.mean(axis=0)
    v = x.var(axis=0)
    return (x - m) / jnp.sqrt(v + 1e-5)


def _node_mlp(h, agg, p, relu_after):
    z = (1.0 + p["eps"]) * h + agg
    t = jax.nn.relu(_bn(z @ p["mlp1"]["w"] + p["mlp1"]["b"]))
    u = _bn(t @ p["mlp2"]["w"] + p["mlp2"]["b"])
    if relu_after:
        u = jax.nn.relu(u)
    return u + h


def kernel(x, edge_attr, params, edge_index, batch):
    n = x.shape[0]
    gkey = jax.random.key(42)
    g_noise = jax.random.gumbel(jax.random.fold_in(gkey, 1), (n, 2), dtype=jnp.float32)
    perm = jax.random.permutation(jax.random.fold_in(gkey, 2), _NG)
    zeros_tile = jnp.zeros((_RPT, _EMB), jnp.float32)
    src = edge_index[0].astype(jnp.int32)
    dst = edge_index[1].astype(jnp.int32)
    padw = _EPT_P - _EPT
    src_p = jnp.concatenate(
        [src.reshape(_NS, _EPT),
         jnp.zeros((_NS, padw), jnp.int32)], axis=1).reshape(-1)
    dst_pad = _N + (jnp.arange(padw, dtype=jnp.int32) % (_NPAD - _N - 1))
    dst_p = jnp.concatenate(
        [dst.reshape(_NS, _EPT),
         jnp.broadcast_to(dst_pad, (_NS, padw))], axis=1).reshape(-1)
    src2 = jnp.concatenate([src_p, src_p + _N])
    dstf = dst_p
    attr_p = jnp.concatenate(
        [edge_attr.reshape(_NS, _EPT, -1),
         jnp.zeros((_NS, padw, edge_attr.shape[1]), jnp.float32)],
        axis=1).reshape(_EPAD, -1)

    enc, rat = params["enc_layers"], params["rat_layers"]
    xf = x @ params["node_enc"]["w"] + params["node_enc"]["b"]

    def e_of(pe, pr):
        return jnp.stack([attr_p @ pe["edge"]["w"] + pe["edge"]["b"],
                          attr_p @ pr["edge"]["w"] + pr["edge"]["b"]])

    agg1 = _edge_stage_pair(jnp.stack([xf, xf]), e_of(enc[0], rat[0]),
                            src2, dstf, zeros_tile)[:, :_N]
    h_enc1 = _node_mlp(xf, agg1[0], enc[0], True)
    h_rat1 = _node_mlp(xf, agg1[1], rat[0], True)
    agg2 = _edge_stage_pair(jnp.stack([h_enc1, h_rat1]), e_of(enc[1], rat[1]),
                            src2, dstf, zeros_tile)[:, :_N]
    h_node = _node_mlp(h_enc1, agg2[0], enc[1], False)
    x_rat = _node_mlp(h_rat1, agg2[1], rat[1], False)

    gl = (jax.nn.relu(_bn(x_rat @ params["gate1"]["w"] + params["gate1"]["b"]))
          @ params["gate2"]["w"] + params["gate2"]["b"])
    gate = jax.nn.softmax(gl + g_noise, axis=-1)[:, -1:]

    onehot = (batch[None, :] == jnp.arange(_NG)[:, None]).astype(jnp.float32)
    V = jnp.concatenate([h_node, gate * h_node, gate,
                         jnp.ones((n, 1), jnp.float32)], axis=1)
    S = onehot @ V
    sum_h, sum_gh = S[:, :_EMB], S[:, _EMB:2 * _EMB]
    r_sum, cnt = S[:, 2 * _EMB:2 * _EMB + 1], S[:, 2 * _EMB + 1:]
    cnt_c = jnp.maximum(cnt, 1.0)
    h_r = sum_gh / cnt_c
    h_env = (sum_h - sum_gh) / cnt_c
    h_out = sum_h / cnt_c
    r_num = r_sum + 1e-8
    e_num = (cnt - r_sum) + 1e-8
    loss_reg = jnp.abs(r_num / (r_num + e_num) - _GAMMA).mean()

    def head(v):
        return (jax.nn.relu(_bn(v @ params["pred1"]["w"] + params["pred1"]["b"]))
                @ params["pred2"]["w"] + params["pred2"]["b"])

    pred_rem = head(h_r)
    T = 0.2
    xa = jnp.linalg.norm(h_r, axis=1)
    aa = jnp.linalg.norm(h_out, axis=1)
    ca = jnp.linalg.norm(h_env, axis=1)
    sim = jnp.exp((h_r @ h_out.T) / (jnp.outer(xa, aa) + 1e-8) / T)
    sim_cp = jnp.exp((h_r @ h_env.T) / (jnp.outer(xa, ca) + 1e-8) / T)
    pos = jnp.diag(sim)
    loss2 = pos / (sim_cp.sum(axis=1) + pos)
    loss_con = -jnp.log(loss2).mean()
    pred_rep = head(h_r + h_env[perm])
    return (pred_rep, pred_rem, loss_reg, loss_con)


# speedup vs baseline: 2.7452x; 1.0936x over previous
import functools

import jax
import jax.numpy as jnp
from jax import lax
from jax.experimental import pallas as pl
from jax.experimental.pallas import tpu as pltpu
from jax.experimental.pallas import tpu_sc as plsc

_EMB = 128
_N = 10000
_E = 320000
_NG = 128
_GAMMA = 0.4
_NS = 16
_CH = 64
_CPT = 320
_EPT = _E // _NS
_EPT_P = _CH * _CPT
_EPAD = _NS * _EPT_P
_NPAD = 10240
_RPT = _NPAD // _NS


def _edge_stage_pair(h_pair, e_pair, src2, dstf, zeros_tile):
    h_flat = h_pair.reshape(2 * _N, _EMB)
    e_flat = e_pair.reshape(2 * _EPAD, _EMB)
    mesh = plsc.VectorSubcoreMesh(core_axis_name="c", subcore_axis_name="s")

    @functools.partial(
        pl.kernel,
        out_type=jax.ShapeDtypeStruct((2, _NPAD, _EMB), jnp.float32),
        mesh=mesh,
        scratch_types=[
            pltpu.VMEM_SHARED((_NPAD, _EMB), jnp.float32),
            pltpu.VMEM((4, _CH), jnp.int32),
            pltpu.VMEM((4, _CH), jnp.int32),
            pltpu.VMEM((2, _CH, _EMB), jnp.float32),
            pltpu.VMEM((2, _CH, _EMB), jnp.float32),
        ] + [pltpu.SemaphoreType.DMA] * 8,
    )
    def k(h_hbm, e_hbm, src_hbm, dst_hbm, z_hbm, out_hbm,
          agg_sh, srcb, dstb, rows_v, e_v,
          si0, si1, si2, si3, sg0, sg1, se0, se1):
        c = lax.axis_index("c")
        s = lax.axis_index("s")
        si = (si0, si1, si2, si3)
        sg = (sg0, sg1)
        se = (se0, se1)
        ebase = c * _EPAD + s * _EPT_P
        dbase = s * _EPT_P

        pltpu.sync_copy(z_hbm, agg_sh.at[pl.ds(s * _RPT, _RPT)])
        plsc.subcore_barrier()

        def idx_issue(g, slot):
            pltpu.async_copy(src_hbm.at[pl.ds(ebase + g * _CH, _CH)],
                             srcb.at[slot], si[slot])
            pltpu.async_copy(dst_hbm.at[pl.ds(dbase + g * _CH, _CH)],
                             dstb.at[slot], si[slot])

        def idx_wait(slot):
            pltpu.make_async_copy(src_hbm.at[pl.ds(0, _CH)],
                                  srcb.at[slot], si[slot]).wait()
            pltpu.make_async_copy(src_hbm.at[pl.ds(0, _CH)],
                                  dstb.at[slot], si[slot]).wait()

        def ge_issue(g, slot, b):
            pltpu.async_copy(h_hbm.at[srcb.at[slot]], rows_v.at[b], sg[b])
            pltpu.async_copy(e_hbm.at[pl.ds(ebase + g * _CH, _CH)],
                             e_v.at[b], se[b])

        def finish(slot, b):
            pltpu.make_async_copy(h_hbm.at[srcb.at[slot]],
                                  rows_v.at[b], sg[b]).wait()
            pltpu.make_async_copy(e_hbm.at[pl.ds(0, _CH)],
                                  e_v.at[b], se[b]).wait()

            def rg_body(r, rc):
                r4 = r * 4
                for rr in range(4):
                    for j in range(_EMB // 16):
                        sl = pl.ds(j * 16, 16)
                        rows_v[b, r4 + rr, sl] = jnp.maximum(
                            rows_v[b, r4 + rr, sl] + e_v[b, r4 + rr, sl], 0.0)
                return rc

            lax.fori_loop(0, _CH // 4, rg_body, 0)
            pltpu.sync_copy(rows_v.at[b], agg_sh.at[dstb.at[slot]], add=True)

        idx_issue(0, 0)
        idx_issue(1, 1)
        idx_wait(0)
        ge_issue(0, 0, 0)

        def body(i, carry):
            g0 = i * 4
            for u in range(4):
                g = g0 + u

                @pl.when(g + 1 < _CPT)
                def _(u=u, g=g):
                    idx_wait((u + 1) % 4)
                    ge_issue(g + 1, (u + 1) % 4, (u + 1) % 2)

                @pl.when(g + 2 < _CPT)
                def _(u=u, g=g):
                    idx_issue(g + 2, (u + 2) % 4)

                finish(u, u % 2)
            return carry

        lax.fori_loop(0, _CPT // 4, body, 0)
        plsc.subcore_barrier()
        pltpu.sync_copy(agg_sh.at[pl.ds(s * _RPT, _RPT)],
                        out_hbm.at[c, pl.ds(s * _RPT, _RPT)])

    return k(h_flat, e_flat, src2, dstf, zeros_tile)


def _bn_in(t):
    m = jnp.mean(t, axis=0, keepdims=True)
    d = t - m
    v = jnp.mean(d * d, axis=0, keepdims=True)
    return d / jnp.sqrt(v + 1e-5)


def _xf_body(x_ref, w_ref, b_ref, o_ref):
    o_ref[...] = jnp.dot(x_ref[...], w_ref[...],
                         preferred_element_type=jnp.float32) + b_ref[...]


def _xf_enc(x, w, b):
    return pl.pallas_call(
        _xf_body,
        out_shape=jax.ShapeDtypeStruct((_N, _EMB), jnp.float32),
    )(x, w, b)


_ECH = 2048


def _edge_tr_body(a_ref, w_ref, b_ref, e_ref):
    a = a_ref[...]
    for c in range(2):
        e_ref[c] = jnp.dot(a, w_ref[c],
                           preferred_element_type=jnp.float32) + b_ref[c]


def _edge_transform(attr_p, we, be):
    return pl.pallas_call(
        _edge_tr_body,
        grid=(_EPAD // _ECH,),
        in_specs=[
            pl.BlockSpec((_ECH, 16), lambda i: (i, 0)),
            pl.BlockSpec((2, 16, _EMB), lambda i: (0, 0, 0)),
            pl.BlockSpec((2, 1, _EMB), lambda i: (0, 0, 0)),
        ],
        out_specs=pl.BlockSpec((2, _ECH, _EMB), lambda i: (0, i, 0)),
        out_shape=jax.ShapeDtypeStruct((2, _EPAD, _EMB), jnp.float32),
    )(attr_p, we, be)


def _node_mlp_body(relu_after, h_ref, agg_ref, eps_ref, w1_ref, b1_ref,
                   w2_ref, b2_ref, out_ref):
    X = h_ref[0]
    z = (1.0 + eps_ref[pl.program_id(0), 0]) * X + agg_ref[0]
    t = jnp.dot(z, w1_ref[0], preferred_element_type=jnp.float32) + b1_ref[0]
    t = jnp.maximum(_bn_in(t), 0.0)
    u = jnp.dot(t, w2_ref[0], preferred_element_type=jnp.float32) + b2_ref[0]
    u = _bn_in(u)
    if relu_after:
        u = jnp.maximum(u, 0.0)
    out_ref[0] = u + X


def _node_mlp_pair(h_pair, agg_pair, eps, w1, b1, w2, b2, relu_after):
    return pl.pallas_call(
        functools.partial(_node_mlp_body, relu_after),
        grid=(2,),
        in_specs=[
            pl.BlockSpec((1, _N, _EMB), lambda i: (i, 0, 0)),
            pl.BlockSpec((1, _N, _EMB), lambda i: (i, 0, 0)),
            pl.BlockSpec(memory_space=pltpu.SMEM),
            pl.BlockSpec((1, _EMB, 2 * _EMB), lambda i: (i, 0, 0)),
            pl.BlockSpec((1, 1, 2 * _EMB), lambda i: (i, 0, 0)),
            pl.BlockSpec((1, 2 * _EMB, _EMB), lambda i: (i, 0, 0)),
            pl.BlockSpec((1, 1, _EMB), lambda i: (i, 0, 0)),
        ],
        out_specs=pl.BlockSpec((1, _N, _EMB), lambda i: (i, 0, 0)),
        out_shape=jax.ShapeDtypeStruct((2, _N, _EMB), jnp.float32),
    )(h_pair, agg_pair, eps, w1, b1, w2, b2)


def _gate_pool_body(xr_ref, h_ref, gn_ref, batch_ref, w1_ref, b1_ref,
                    w2_ref, b2_ref, s1_ref, s2_ref, s3_ref):
    t = jnp.dot(xr_ref[...], w1_ref[...],
                preferred_element_type=jnp.float32) + b1_ref[...]
    t = jnp.maximum(_bn_in(t), 0.0)
    logits = jnp.dot(t, w2_ref[...],
                     preferred_element_type=jnp.float32) + b2_ref[...]
    ln = logits[:, :2] + gn_ref[...]
    m = jnp.max(ln, axis=1, keepdims=True)
    p = jnp.exp(ln - m)
    gate = p[:, 1:2] / (p[:, 0:1] + p[:, 1:2])
    onehot = (batch_ref[...] == lax.broadcasted_iota(jnp.int32, (_NG, _N), 0)
              ).astype(jnp.float32)
    h = h_ref[...]
    s1_ref[...] = jnp.dot(onehot, h, preferred_element_type=jnp.float32)
    s2_ref[...] = jnp.dot(onehot, gate * h, preferred_element_type=jnp.float32)
    r_sum = jnp.dot(onehot, gate, preferred_element_type=jnp.float32)
    cnt = jnp.sum(onehot, axis=1, keepdims=True)
    s3_ref[...] = jnp.concatenate(
        [r_sum, cnt, jnp.zeros((_NG, 6), jnp.float32)], axis=1)


def _gate_pool(x_rat, h_node, g_noise, batch2d, w1, b1, w2p, b2p):
    return pl.pallas_call(
        _gate_pool_body,
        out_shape=(jax.ShapeDtypeStruct((_NG, _EMB), jnp.float32),
                   jax.ShapeDtypeStruct((_NG, _EMB), jnp.float32),
                   jax.ShapeDtypeStruct((_NG, 8), jnp.float32)),
    )(x_rat, h_node, g_noise, batch2d, w1, b1, w2p, b2p)


def _tail_body(s1_ref, s2_ref, s3_ref, permm_ref, p1w_ref, p1b_ref,
               p2w_ref, p2b_ref, rep_ref, rem_ref, reg_ref, con_ref):
    S1, S2 = s1_ref[...], s2_ref[...]
    r_sum = s3_ref[:, 0:1]
    cnt = s3_ref[:, 1:2]
    cnt_c = jnp.maximum(cnt, 1.0)
    h_r = S2 / cnt_c
    h_env = (S1 - S2) / cnt_c
    h_out = S1 / cnt_c
    r_num = r_sum + 1e-8
    e_num = (cnt - r_sum) + 1e-8
    reg_ref[...] = jnp.mean(jnp.abs(r_num / (r_num + e_num) - _GAMMA))[None, None]

    def head(v):
        t = jnp.dot(v, p1w_ref[...],
                    preferred_element_type=jnp.float32) + p1b_ref[...]
        t = jnp.maximum(_bn_in(t), 0.0)
        return jnp.dot(t, p2w_ref[...],
                       preferred_element_type=jnp.float32) + p2b_ref[...]

    rem_ref[...] = head(h_r)
    comb = h_r + jnp.dot(permm_ref[...], h_env,
                         preferred_element_type=jnp.float32)
    rep_ref[...] = head(comb)

    def norms(v):
        return jnp.sqrt(jnp.sum(v * v, axis=1, keepdims=True))

    T = 0.2
    xa, aa, ca = norms(h_r), norms(h_out), norms(h_env)

    def dot_t(a, b):
        return lax.dot_general(a, b, (((1,), (1,)), ((), ())),
                               preferred_element_type=jnp.float32)

    sim = jnp.exp(dot_t(h_r, h_out) / (xa * aa.T + 1e-8) / T)
    sim_cp = jnp.exp(dot_t(h_r, h_env) / (xa * ca.T + 1e-8) / T)
    eye = (lax.broadcasted_iota(jnp.int32, (_NG, _NG), 0) ==
           lax.broadcasted_iota(jnp.int32, (_NG, _NG), 1)).astype(jnp.float32)
    pos = jnp.sum(sim * eye, axis=1, keepdims=True)
    loss2 = pos / (jnp.sum(sim_cp, axis=1, keepdims=True) + pos)
    con_ref[...] = (-jnp.mean(jnp.log(loss2)))[None, None]


def _tail(S1, S2, S3, permm, p1w, p1b, p2w, p2b):
    return pl.pallas_call(
        _tail_body,
        out_shape=(jax.ShapeDtypeStruct((_NG, 10), jnp.float32),
                   jax.ShapeDtypeStruct((_NG, 10), jnp.float32),
                   jax.ShapeDtypeStruct((1, 1), jnp.float32),
                   jax.ShapeDtypeStruct((1, 1), jnp.float32)),
    )(S1, S2, S3, permm, p1w, p1b, p2w, p2b)


def _bn(x):
    m = x.mean(axis=0)
    v = x.var(axis=0)
    return (x - m) / jnp.sqrt(v + 1e-5)


def _node_mlp(h, agg, p, relu_after):
    z = (1.0 + p["eps"]) * h + agg
    t = jax.nn.relu(_bn(z @ p["mlp1"]["w"] + p["mlp1"]["b"]))
    u = _bn(t @ p["mlp2"]["w"] + p["mlp2"]["b"])
    if relu_after:
        u = jax.nn.relu(u)
    return u + h


def kernel(x, edge_attr, params, edge_index, batch):
    n = x.shape[0]
    gkey = jax.random.key(42)
    g_noise = jax.random.gumbel(jax.random.fold_in(gkey, 1), (n, 2), dtype=jnp.float32)
    perm = jax.random.permutation(jax.random.fold_in(gkey, 2), _NG)
    zeros_tile = jnp.zeros((_RPT, _EMB), jnp.float32)
    src = edge_index[0].astype(jnp.int32)
    dst = edge_index[1].astype(jnp.int32)
    padw = _EPT_P - _EPT
    src_p = jnp.concatenate(
        [src.reshape(_NS, _EPT),
         jnp.zeros((_NS, padw), jnp.int32)], axis=1).reshape(-1)
    dst_pad = _N + (jnp.arange(padw, dtype=jnp.int32) % (_NPAD - _N - 1))
    dst_p = jnp.concatenate(
        [dst.reshape(_NS, _EPT),
         jnp.broadcast_to(dst_pad, (_NS, padw))], axis=1).reshape(-1)
    src2 = jnp.concatenate([src_p, src_p + _N])
    dstf = dst_p
    attr_p = jnp.concatenate(
        [edge_attr.reshape(_NS, _EPT, -1),
         jnp.zeros((_NS, padw, edge_attr.shape[1]), jnp.float32)],
        axis=1).reshape(_EPAD, -1)

    enc, rat = params["enc_layers"], params["rat_layers"]
    xf = _xf_enc(x, params["node_enc"]["w"], params["node_enc"]["b"][None, :])

    def e_of(pe, pr):
        we = jnp.stack([pe["edge"]["w"], pr["edge"]["w"]])
        be = jnp.stack([pe["edge"]["b"][None, :], pr["edge"]["b"][None, :]])
        return _edge_transform(attr_p, we, be)

    def mlp_pair(h_pair, agg_pair, pe, pr, relu_after):
        eps = jnp.stack([pe["eps"], pr["eps"]]).reshape(2, 1)
        w1 = jnp.stack([pe["mlp1"]["w"], pr["mlp1"]["w"]])
        b1 = jnp.stack([pe["mlp1"]["b"][None, :], pr["mlp1"]["b"][None, :]])
        w2 = jnp.stack([pe["mlp2"]["w"], pr["mlp2"]["w"]])
        b2 = jnp.stack([pe["mlp2"]["b"][None, :], pr["mlp2"]["b"][None, :]])
        return _node_mlp_pair(h_pair, agg_pair, eps, w1, b1, w2, b2, relu_after)

    h0 = jnp.stack([xf, xf])
    agg1 = _edge_stage_pair(h0, e_of(enc[0], rat[0]), src2, dstf, zeros_tile)
    h1 = mlp_pair(h0, agg1, enc[0], rat[0], True)
    agg2 = _edge_stage_pair(h1, e_of(enc[1], rat[1]), src2, dstf, zeros_tile)
    h2 = mlp_pair(h1, agg2, enc[1], rat[1], False)
    h_node, x_rat = h2[0], h2[1]

    gw2p = jnp.concatenate(
        [params["gate2"]["w"], jnp.zeros((2 * _EMB, _EMB - 2), jnp.float32)],
        axis=1)
    gb2p = jnp.concatenate(
        [params["gate2"]["b"], jnp.zeros((_EMB - 2,), jnp.float32)])[None, :]
    S1, S2, S3 = _gate_pool(x_rat, h_node, g_noise,
                            batch.astype(jnp.int32)[None, :],
                            params["gate1"]["w"], params["gate1"]["b"][None, :],
                            gw2p, gb2p)
    permm = jax.nn.one_hot(perm, _NG, dtype=jnp.float32)
    rep, rem, reg, con = _tail(S1, S2, S3, permm,
                               params["pred1"]["w"], params["pred1"]["b"][None, :],
                               params["pred2"]["w"], params["pred2"]["b"][None, :])
    return (rep, rem, reg[0, 0], con[0, 0])
